# baseline (speedup 1.0000x reference)
import functools

import jax
import jax.numpy as jnp
from jax import lax
from jax.experimental import pallas as pl
from jax.experimental.pallas import tpu as pltpu
from jax.experimental.pallas import tpu_sc as plsc

B, D1, D2 = 64, 2048, 512

L = 16
NC, NS = 2, 16
NW = NC * NS
KSC = 8
HROWS = D1 // 2
RT = 64
NT = HROWS // RT
NCH = D2 // L
UC = 4
BPS = KSC // NC

SG = 32
NG = D1 // SG


def _build_sc_argmin():
    mesh = plsc.VectorSubcoreMesh(
        core_axis_name="c", subcore_axis_name="s", num_cores=NC, num_subcores=NS
    )

    @functools.partial(
        pl.kernel,
        mesh=mesh,
        out_type=jax.ShapeDtypeStruct((KSC, D2), jnp.int32),
        scratch_types=[
            pltpu.VMEM((2, RT, D2), jnp.float32),
            pltpu.VMEM((NCH, UC, L), jnp.float32),
            pltpu.VMEM((NCH, UC, L), jnp.int32),
            pltpu.VMEM((D2,), jnp.float32),
            pltpu.VMEM((D2,), jnp.int32),
            pltpu.VMEM((D2,), jnp.float32),
            pltpu.VMEM((D2,), jnp.int32),
            pltpu.VMEM_SHARED((BPS, 2, D2), jnp.float32),
            pltpu.VMEM_SHARED((BPS, 2, D2), jnp.int32),
            pltpu.SemaphoreType.DMA,
            pltpu.SemaphoreType.DMA,
        ],
    )
    def sc_argmin(
        x_hbm, out_hbm, buf, accv, accg, mval, acci, pv, pi, shv, shi,
        sem0, sem1
    ):
        core = lax.axis_index("c")
        sub = lax.axis_index("s")
        pair = sub // 2
        half = sub % 2
        b = core * BPS + pair
        row0 = half * HROWS
        g0base = half * (HROWS // UC)
        sems = (sem0, sem1)
        active = pair < BPS

        def init_accs():
            for c in range(NCH):
                for u in range(UC):
                    accv[c, u] = jnp.full((L,), jnp.inf, jnp.float32)
                    accg[c, u] = jnp.zeros((L,), jnp.int32)

        def start_copy(t, slot):
            pltpu.make_async_copy(
                x_hbm.at[b, pl.ds(row0 + t * RT, RT), :],
                buf.at[slot],
                sems[slot],
            ).start()

        def wait_copy(t, slot):
            pltpu.make_async_copy(
                x_hbm.at[b, pl.ds(row0 + t * RT, RT), :],
                buf.at[slot],
                sems[slot],
            ).wait()

        def consume(slot, t):
            for c in range(NCH):
                chains = [(accv[c, u], accg[c, u]) for u in range(UC)]
                gi0 = g0base + jnp.full((L,), t * (RT // UC), jnp.int32)

                def grp_body(g, carry, _c=c, _slot=slot):
                    a0, g0, a1, g1, a2, g2, a3, g3, gi = carry
                    res = []
                    for u, (au, gu) in enumerate(
                        ((a0, g0), (a1, g1), (a2, g2), (a3, g3))
                    ):
                        v = buf[_slot, g * UC + u, pl.ds(_c * L, L)]
                        lt = v < au
                        res.append(jnp.where(lt, v, au))
                        res.append(jnp.where(lt, gi, gu))
                    res.append(gi + 1)
                    return tuple(res)

                init = tuple(x for ch in chains for x in ch) + (gi0,)
                out = lax.fori_loop(0, RT // UC, grp_body, init, unroll=4)
                for u in range(UC):
                    accv[c, u] = out[2 * u]
                    accg[c, u] = out[2 * u + 1]

        @pl.when(active)
        def _():
            init_accs()
            start_copy(0, 0)

            def tile_pair(tp, _):
                t0 = tp * 2
                start_copy(t0 + 1, 1)
                wait_copy(t0, 0)
                consume(0, t0)

                @pl.when(t0 + 2 < NT)
                def _():
                    start_copy(t0 + 2, 0)

                wait_copy(t0 + 1, 1)
                consume(1, t0 + 1)
                return 0

            lax.fori_loop(0, NT // 2, tile_pair, 0)

            for c in range(NCH):
                mv = accv[c, 0]
                mr = accg[c, 0] * UC
                for u in range(1, UC):
                    av = accv[c, u]
                    ar = accg[c, u] * UC + u
                    take = (av < mv) | ((av == mv) & (ar < mr))
                    mv = jnp.where(take, av, mv)
                    mr = jnp.where(take, ar, mr)
                mval[pl.ds(c * L, L)] = mv
                acci[pl.ds(c * L, L)] = mr

            pltpu.sync_copy(mval, shv.at[pair, half])
            pltpu.sync_copy(acci, shi.at[pair, half])

        plsc.subcore_barrier()

        @pl.when(active & (half == 0))
        def _():
            pltpu.sync_copy(shv.at[pair, 1], pv)
            pltpu.sync_copy(shi.at[pair, 1], pi)
            for c in range(NCH):
                av = mval[pl.ds(c * L, L)]
                ai = acci[pl.ds(c * L, L)]
                bv = pv[pl.ds(c * L, L)]
                bi = pi[pl.ds(c * L, L)]
                lt = bv < av
                acci[pl.ds(c * L, L)] = jnp.where(lt, bi, ai)
            pltpu.sync_copy(acci, out_hbm.at[b])

    return sc_argmin


def _tc_body(x_ref, o_ref):
    def group_body(i, carry):
        mv, mi = carry
        v = x_ref[0, pl.ds(i * SG, SG), :]
        lt = v < mv
        gi = jnp.full((SG, D2), i, jnp.int32)
        return jnp.where(lt, v, mv), jnp.where(lt, gi, mi)

    mv0 = jnp.full((SG, D2), jnp.inf, jnp.float32)
    mi0 = jnp.zeros((SG, D2), jnp.int32)
    mv, mi = lax.fori_loop(0, NG, group_body, (mv0, mi0), unroll=4)

    sub = lax.broadcasted_iota(jnp.int32, (SG, D2), 0)
    row = mi * SG + sub

    sh = SG // 2
    while sh >= 1:
        mv2 = jnp.roll(mv, -sh, axis=0)
        row2 = jnp.roll(row, -sh, axis=0)
        take = (mv2 < mv) | ((mv2 == mv) & (row2 < row))
        mv = jnp.where(take, mv2, mv)
        row = jnp.where(take, row2, row)
        sh //= 2

    o_ref[0, 0] = row[0, :]


_tc_argmin_3d = pl.pallas_call(
    _tc_body,
    grid=(B - KSC,),
    in_specs=[pl.BlockSpec((1, D1, D2), lambda i: (i + KSC, 0, 0))],
    out_specs=pl.BlockSpec((1, 1, D2), lambda i: (i, 0, 0)),
    out_shape=jax.ShapeDtypeStruct((B - KSC, 1, D2), jnp.int32),
)

_sc_argmin = _build_sc_argmin()


@jax.jit
def kernel(x):
    sc_out = _sc_argmin(x)
    tc_out = _tc_argmin_3d(x)[:, 0, :]
    return jnp.concatenate([sc_out, tc_out], axis=0).astype(jnp.int64)


# speedup vs baseline: 1.0442x; 1.0442x over previous
import functools

import jax
import jax.numpy as jnp
from jax import lax
from jax.experimental import pallas as pl
from jax.experimental.pallas import tpu as pltpu
from jax.experimental.pallas import tpu_sc as plsc

B, D1, D2 = 64, 2048, 512
L = 16
NC, NS = 2, 16
NW = NC * NS
KSC = 16
HALF = D2 // 2
RT = 64
NT = D1 // RT
NCH = HALF // L
SG = 32
NG = D1 // SG


def _build_sc_argmin():
    mesh = plsc.VectorSubcoreMesh(
        core_axis_name="c", subcore_axis_name="s", num_cores=NC, num_subcores=NS
    )

    @functools.partial(
        pl.kernel,
        mesh=mesh,
        out_type=jax.ShapeDtypeStruct((KSC, D2), jnp.int32),
        scratch_types=[
            pltpu.VMEM((2, RT, HALF), jnp.float32),
            pltpu.VMEM((HALF,), jnp.float32),
            pltpu.VMEM((HALF,), jnp.int32),
            pltpu.SemaphoreType.DMA,
            pltpu.SemaphoreType.DMA,
        ],
    )
    def sc_argmin(x_hbm, out_hbm, buf, accv, acci, sem0, sem1):
        wid = lax.axis_index("s") * NC + lax.axis_index("c")
        b = wid // 2
        col0 = (wid % 2) * HALF
        sems = (sem0, sem1)

        for c in range(NCH):
            accv[pl.ds(c * L, L)] = jnp.full((L,), jnp.inf, jnp.float32)
            acci[pl.ds(c * L, L)] = jnp.zeros((L,), jnp.int32)

        def start_copy(t, slot):
            pltpu.make_async_copy(
                x_hbm.at[b, pl.ds(t * RT, RT), pl.ds(col0, HALF)],
                buf.at[slot],
                sems[slot],
            ).start()

        def wait_copy(t, slot):
            pltpu.make_async_copy(
                x_hbm.at[b, pl.ds(t * RT, RT), pl.ds(col0, HALF)],
                buf.at[slot],
                sems[slot],
            ).wait()

        def consume(slot, base_row):
            base = jnp.full((L,), base_row, jnp.int32)
            for c in range(NCH):
                av = accv[pl.ds(c * L, L)]
                ai = acci[pl.ds(c * L, L)]

                def row_body(r, carry, _c=c, _slot=slot):
                    av, ai, ridx = carry
                    v = buf[_slot, r, pl.ds(_c * L, L)]
                    lt = v < av
                    return (
                        jnp.where(lt, v, av),
                        jnp.where(lt, ridx, ai),
                        ridx + 1,
                    )

                av, ai, _ = lax.fori_loop(
                    0, RT, row_body, (av, ai, base), unroll=8
                )
                accv[pl.ds(c * L, L)] = av
                acci[pl.ds(c * L, L)] = ai

        start_copy(0, 0)

        def tile_pair(tp, _):
            t0 = tp * 2
            start_copy(t0 + 1, 1)
            wait_copy(t0, 0)
            consume(0, t0 * RT)

            @pl.when(t0 + 2 < NT)
            def _():
                start_copy(t0 + 2, 0)

            wait_copy(t0 + 1, 1)
            consume(1, (t0 + 1) * RT)
            return 0

        lax.fori_loop(0, NT // 2, tile_pair, 0)
        pltpu.sync_copy(acci, out_hbm.at[b, pl.ds(col0, HALF)])

    return sc_argmin


def _tc_body(x_ref, o_ref):
    def group_body(i, carry):
        mv, mi = carry
        v = x_ref[0, pl.ds(i * SG, SG), :]
        lt = v < mv
        gi = jnp.full((SG, D2), i, jnp.int32)
        return jnp.where(lt, v, mv), jnp.where(lt, gi, mi)

    mv0 = jnp.full((SG, D2), jnp.inf, jnp.float32)
    mi0 = jnp.zeros((SG, D2), jnp.int32)
    mv, mi = lax.fori_loop(0, NG, group_body, (mv0, mi0), unroll=4)

    sub = lax.broadcasted_iota(jnp.int32, (SG, D2), 0)
    row = mi * SG + sub

    sh = SG // 2
    while sh >= 1:
        mv2 = jnp.roll(mv, -sh, axis=0)
        row2 = jnp.roll(row, -sh, axis=0)
        take = (mv2 < mv) | ((mv2 == mv) & (row2 < row))
        mv = jnp.where(take, mv2, mv)
        row = jnp.where(take, row2, row)
        sh //= 2

    o_ref[0, 0] = row[0, :]


_tc_argmin_3d = pl.pallas_call(
    _tc_body,
    grid=(B - KSC,),
    in_specs=[pl.BlockSpec((1, D1, D2), lambda i: (i + KSC, 0, 0))],
    out_specs=pl.BlockSpec((1, 1, D2), lambda i: (i, 0, 0)),
    out_shape=jax.ShapeDtypeStruct((B - KSC, 1, D2), jnp.int32),
)


def _tc_argmin(x):
    return _tc_argmin_3d(x)[:, 0, :]

_sc_argmin = _build_sc_argmin()


@jax.jit
def kernel(x):
    sc_out = _sc_argmin(x)
    tc_out = _tc_argmin(x)
    return jnp.concatenate([sc_out, tc_out], axis=0).astype(jnp.int64)


# speedup vs baseline: 1.0533x; 1.0087x over previous
import functools

import jax
import jax.numpy as jnp
from jax import lax
from jax.experimental import pallas as pl
from jax.experimental.pallas import tpu as pltpu
from jax.experimental.pallas import tpu_sc as plsc

B, D1, D2 = 64, 2048, 512
L = 16
NC, NS = 2, 16
NW = NC * NS
KSC = 16
HALF = D2 // 2
RT = 64
NT = D1 // RT
NCH = HALF // L
SG = 32
NG = D1 // SG


def _build_sc_argmin():
    mesh = plsc.VectorSubcoreMesh(
        core_axis_name="c", subcore_axis_name="s", num_cores=NC, num_subcores=NS
    )

    @functools.partial(
        pl.kernel,
        mesh=mesh,
        out_type=jax.ShapeDtypeStruct((KSC, D2), jnp.int32),
        scratch_types=[
            pltpu.VMEM((2, RT, HALF), jnp.float32),
            pltpu.VMEM((HALF,), jnp.float32),
            pltpu.VMEM((HALF,), jnp.int32),
            pltpu.SemaphoreType.DMA,
            pltpu.SemaphoreType.DMA,
        ],
    )
    def sc_argmin(x_hbm, out_hbm, buf, accv, acci, sem0, sem1):
        wid = lax.axis_index("s") * NC + lax.axis_index("c")
        b = wid // 2
        col0 = (wid % 2) * HALF
        sems = (sem0, sem1)

        for c in range(NCH):
            accv[pl.ds(c * L, L)] = jnp.full((L,), jnp.inf, jnp.float32)
            acci[pl.ds(c * L, L)] = jnp.zeros((L,), jnp.int32)

        def start_copy(t, slot):
            pltpu.make_async_copy(
                x_hbm.at[b, pl.ds(t * RT, RT), pl.ds(col0, HALF)],
                buf.at[slot],
                sems[slot],
            ).start()

        def wait_copy(t, slot):
            pltpu.make_async_copy(
                x_hbm.at[b, pl.ds(t * RT, RT), pl.ds(col0, HALF)],
                buf.at[slot],
                sems[slot],
            ).wait()

        def consume(slot, base_row):
            base = jnp.full((L,), base_row, jnp.int32)
            for c in range(NCH):
                av = accv[pl.ds(c * L, L)]
                ai = acci[pl.ds(c * L, L)]

                def row_body(r, carry, _c=c, _slot=slot):
                    av, ai, ridx = carry
                    v = buf[_slot, r, pl.ds(_c * L, L)]
                    lt = v < av
                    return (
                        jnp.where(lt, v, av),
                        jnp.where(lt, ridx, ai),
                        ridx + 1,
                    )

                av, ai, _ = lax.fori_loop(
                    0, RT, row_body, (av, ai, base), unroll=8
                )
                accv[pl.ds(c * L, L)] = av
                acci[pl.ds(c * L, L)] = ai

        start_copy(0, 0)

        def tile_pair(tp, _):
            t0 = tp * 2
            start_copy(t0 + 1, 1)
            wait_copy(t0, 0)
            consume(0, t0 * RT)

            @pl.when(t0 + 2 < NT)
            def _():
                start_copy(t0 + 2, 0)

            wait_copy(t0 + 1, 1)
            consume(1, (t0 + 1) * RT)
            return 0

        lax.fori_loop(0, NT // 2, tile_pair, 0)
        pltpu.sync_copy(acci, out_hbm.at[b, pl.ds(col0, HALF)])

    return sc_argmin


def _tc_body(x_ref, o_ref):
    def group_body(i, carry):
        mv, mi = carry
        v = x_ref[0, pl.ds(i * SG, SG), :]
        lt = v < mv
        gi = jnp.full((SG, D2), i, jnp.int32)
        return jnp.where(lt, v, mv), jnp.where(lt, gi, mi)

    mv0 = jnp.full((SG, D2), jnp.inf, jnp.float32)
    mi0 = jnp.zeros((SG, D2), jnp.int32)
    mv, mi = lax.fori_loop(0, NG, group_body, (mv0, mi0), unroll=8)

    sub = lax.broadcasted_iota(jnp.int32, (SG, D2), 0)
    row = mi * SG + sub

    sh = SG // 2
    while sh >= 1:
        mv2 = jnp.roll(mv, -sh, axis=0)
        row2 = jnp.roll(row, -sh, axis=0)
        take = (mv2 < mv) | ((mv2 == mv) & (row2 < row))
        mv = jnp.where(take, mv2, mv)
        row = jnp.where(take, row2, row)
        sh //= 2

    o_ref[0, 0] = row[0, :]


_tc_argmin_3d = pl.pallas_call(
    _tc_body,
    grid=(B - KSC,),
    in_specs=[pl.BlockSpec((1, D1, D2), lambda i: (i + KSC, 0, 0))],
    out_specs=pl.BlockSpec((1, 1, D2), lambda i: (i, 0, 0)),
    out_shape=jax.ShapeDtypeStruct((B - KSC, 1, D2), jnp.int32),
)


def _tc_argmin(x):
    return _tc_argmin_3d(x)[:, 0, :]

_sc_argmin = _build_sc_argmin()


@jax.jit
def kernel(x):
    sc_out = _sc_argmin(x)
    tc_out = _tc_argmin(x)
    return jnp.concatenate([sc_out, tc_out], axis=0).astype(jnp.int64)


# speedup vs baseline: 1.0549x; 1.0015x over previous
import functools

import jax
import jax.numpy as jnp
from jax import lax
from jax.experimental import pallas as pl
from jax.experimental.pallas import tpu as pltpu
from jax.experimental.pallas import tpu_sc as plsc

B, D1, D2 = 64, 2048, 512
L = 16
NC, NS = 2, 16
NW = NC * NS
KSC = 16
HALF = D2 // 2
RT = 64
NT = D1 // RT
NCH = HALF // L
SG = 32
NG = D1 // SG


def _build_sc_argmin():
    mesh = plsc.VectorSubcoreMesh(
        core_axis_name="c", subcore_axis_name="s", num_cores=NC, num_subcores=NS
    )

    @functools.partial(
        pl.kernel,
        mesh=mesh,
        out_type=jax.ShapeDtypeStruct((KSC, D2), jnp.int32),
        scratch_types=[
            pltpu.VMEM((2, RT, HALF), jnp.float32),
            pltpu.VMEM((HALF,), jnp.float32),
            pltpu.VMEM((HALF,), jnp.int32),
            pltpu.SemaphoreType.DMA,
            pltpu.SemaphoreType.DMA,
        ],
    )
    def sc_argmin(x_hbm, out_hbm, buf, accv, acci, sem0, sem1):
        wid = lax.axis_index("s") * NC + lax.axis_index("c")
        b = wid // 2
        col0 = (wid % 2) * HALF
        sems = (sem0, sem1)

        for c in range(NCH):
            accv[pl.ds(c * L, L)] = jnp.full((L,), jnp.inf, jnp.float32)
            acci[pl.ds(c * L, L)] = jnp.zeros((L,), jnp.int32)

        def start_copy(t, slot):
            pltpu.make_async_copy(
                x_hbm.at[b, pl.ds(t * RT, RT), pl.ds(col0, HALF)],
                buf.at[slot],
                sems[slot],
            ).start()

        def wait_copy(t, slot):
            pltpu.make_async_copy(
                x_hbm.at[b, pl.ds(t * RT, RT), pl.ds(col0, HALF)],
                buf.at[slot],
                sems[slot],
            ).wait()

        def consume(slot, base_row):
            base = jnp.full((L,), base_row, jnp.int32)
            for c in range(NCH):
                av = accv[pl.ds(c * L, L)]
                ai = acci[pl.ds(c * L, L)]

                def row_body(r, carry, _c=c, _slot=slot):
                    av, ai, ridx = carry
                    v = buf[_slot, r, pl.ds(_c * L, L)]
                    lt = v < av
                    return (
                        jnp.where(lt, v, av),
                        jnp.where(lt, ridx, ai),
                        ridx + 1,
                    )

                av, ai, _ = lax.fori_loop(
                    0, RT, row_body, (av, ai, base), unroll=8
                )
                accv[pl.ds(c * L, L)] = av
                acci[pl.ds(c * L, L)] = ai

        start_copy(0, 0)

        def tile_pair(tp, _):
            t0 = tp * 2
            start_copy(t0 + 1, 1)
            wait_copy(t0, 0)
            consume(0, t0 * RT)

            @pl.when(t0 + 2 < NT)
            def _():
                start_copy(t0 + 2, 0)

            wait_copy(t0 + 1, 1)
            consume(1, (t0 + 1) * RT)
            return 0

        lax.fori_loop(0, NT // 2, tile_pair, 0)
        pltpu.sync_copy(acci, out_hbm.at[b, pl.ds(col0, HALF)])

    return sc_argmin


def _tc_body(x_ref, o_ref):
    def group_body(i, carry):
        mv, mi = carry
        v = x_ref[0, pl.ds(i * SG, SG), :]
        lt = v < mv
        gi = jnp.full((SG, D2), i, jnp.int32)
        return jnp.where(lt, v, mv), jnp.where(lt, gi, mi)

    mv0 = jnp.full((SG, D2), jnp.inf, jnp.float32)
    mi0 = jnp.zeros((SG, D2), jnp.int32)
    mv, mi = lax.fori_loop(0, NG, group_body, (mv0, mi0), unroll=16)

    sub = lax.broadcasted_iota(jnp.int32, (SG, D2), 0)
    row = mi * SG + sub

    sh = SG // 2
    while sh >= 1:
        mv2 = jnp.roll(mv, -sh, axis=0)
        row2 = jnp.roll(row, -sh, axis=0)
        take = (mv2 < mv) | ((mv2 == mv) & (row2 < row))
        mv = jnp.where(take, mv2, mv)
        row = jnp.where(take, row2, row)
        sh //= 2

    o_ref[0, 0] = row[0, :]


_tc_argmin_3d = pl.pallas_call(
    _tc_body,
    grid=(B - KSC,),
    in_specs=[pl.BlockSpec((1, D1, D2), lambda i: (i + KSC, 0, 0))],
    out_specs=pl.BlockSpec((1, 1, D2), lambda i: (i, 0, 0)),
    out_shape=jax.ShapeDtypeStruct((B - KSC, 1, D2), jnp.int32),
)


def _tc_argmin(x):
    return _tc_argmin_3d(x)[:, 0, :]

_sc_argmin = _build_sc_argmin()


@jax.jit
def kernel(x):
    sc_out = _sc_argmin(x)
    tc_out = _tc_argmin(x)
    return jnp.concatenate([sc_out, tc_out], axis=0).astype(jnp.int64)
